# Initial kernel scaffold; baseline (speedup 1.0000x reference)
import jax
import jax.numpy as jnp
from jax.experimental import pallas as pl


def kernel(user_emb, item_emb, u2u_idx, u2u_val, u2i_idx, u2i_val, i2u_idx, i2u_val, i2i_idx, i2i_val):
    raise NotImplementedError("write your pallas kernel here")


# speedup vs baseline: 2.3938x; 2.3938x over previous
import functools

import jax
import jax.numpy as jnp
from jax import lax
from jax.experimental import pallas as pl
from jax.experimental.pallas import tpu as pltpu
from jax.experimental.pallas import tpu_sc as plsc

NU = 10000
NI = 10000
D = 256
E = 160000
HALF = 128
K = 128
NCHUNK = E // K
NSUB = 16
PN = 10240
ROWS_PER_TEC = PN // NSUB


def _to_half_layout(x):
    n = x.shape[0]
    h = x.reshape(n, 2, HALF).transpose(1, 0, 2)
    h = jnp.pad(h, ((0, 0), (0, PN - n), (0, 0)))
    return h.reshape(2 * PN, HALF)


def _from_half_layout(x, n):
    return x.reshape(2, PN, HALF)[:, :n].transpose(1, 0, 2).reshape(n, D)


def _prep_edges(idx, val):
    idx = idx.astype(jnp.int32)
    src = idx[1]
    dst = idx[0]
    gidx = jnp.stack([src, src + PN]).reshape(2, NCHUNK, 1, K)
    didx = dst.reshape(NCHUNK, 1, K)
    vals = val.reshape(NCHUNK, 1, K)
    return gidx, didx, vals


def _layer_body(uL, iL,
                g1, d1, v1,
                g2, d2, v2,
                g3, d3, v3,
                g4, d4, v4,
                u_out, i_out,
                acc, gidx_v, didx_v, val_v, rows_v, zbuf):
    c = lax.axis_index("c")
    s = lax.axis_index("s")

    def zero_zbuf(i, carry):
        for q in range(8):
            zbuf[i, pl.ds(q * 16, 16)] = jnp.zeros((16,), jnp.float32)
        return carry

    lax.fori_loop(0, 128, zero_zbuf, 0)

    def zero_own_rows():
        for k in range(5):
            pltpu.sync_copy(zbuf, acc.at[pl.ds(s * ROWS_PER_TEC + k * 128, 128)])

    def spmm(g, d, v, table):
        def chunk(w, carry):
            t = s + NSUB * w

            @pl.when(t < NCHUNK)
            def _():
                pltpu.sync_copy(g.at[c, t, 0], gidx_v)
                pltpu.sync_copy(d.at[t, 0], didx_v)
                pltpu.sync_copy(v.at[t, 0], val_v)
                pltpu.sync_copy(table.at[gidx_v], rows_v)

                def scale(jb, carry2):
                    vv = val_v[pl.ds(jb * 16, 16)]
                    for l in range(16):
                        vj = vv[l]
                        j = jb * 16 + l
                        for q in range(8):
                            sl = pl.ds(q * 16, 16)
                            rows_v[j, sl] = rows_v[j, sl] * vj
                    return carry2

                lax.fori_loop(0, K // 16, scale, 0)
                pltpu.sync_copy(rows_v, acc.at[didx_v], add=True)

            return carry

        lax.fori_loop(0, (NCHUNK + NSUB - 1) // NSUB, chunk, 0)

    def flush(out_ref):
        pltpu.sync_copy(acc.at[pl.ds(s * ROWS_PER_TEC, ROWS_PER_TEC)],
                        out_ref.at[pl.ds(c * PN + s * ROWS_PER_TEC, ROWS_PER_TEC)])

    zero_own_rows()
    plsc.subcore_barrier()
    spmm(g1, d1, v1, uL)
    spmm(g2, d2, v2, iL)
    plsc.subcore_barrier()
    flush(u_out)
    zero_own_rows()
    plsc.subcore_barrier()
    spmm(g3, d3, v3, iL)
    spmm(g4, d4, v4, uL)
    plsc.subcore_barrier()
    flush(i_out)


_layer_call = functools.partial(
    pl.kernel,
    mesh=plsc.VectorSubcoreMesh(core_axis_name="c", subcore_axis_name="s"),
    out_type=[
        jax.ShapeDtypeStruct((2 * PN, HALF), jnp.float32),
        jax.ShapeDtypeStruct((2 * PN, HALF), jnp.float32),
    ],
    scratch_types=[
        pltpu.VMEM_SHARED((PN, HALF), jnp.float32),
        pltpu.VMEM((K,), jnp.int32),
        pltpu.VMEM((K,), jnp.int32),
        pltpu.VMEM((K,), jnp.float32),
        pltpu.VMEM((K, HALF), jnp.float32),
        pltpu.VMEM((128, HALF), jnp.float32),
    ],
)(_layer_body)


@jax.jit
def kernel(user_emb, item_emb, u2u_idx, u2u_val, u2i_idx, u2i_val,
           i2u_idx, i2u_val, i2i_idx, i2i_val):
    uL = _to_half_layout(user_emb)
    iL = _to_half_layout(item_emb)

    gA, dA, vA = _prep_edges(u2u_idx, u2u_val)
    gB, dB, vB = _prep_edges(u2i_idx, u2i_val)
    gC, dC, vC = _prep_edges(i2i_idx, i2i_val)
    gD, dD, vD = _prep_edges(i2u_idx, i2u_val)

    u1L, i1L = _layer_call(uL, iL, gA, dA, vA, gB, dB, vB,
                           gC, dC, vC, gD, dD, vD)
    u2L, i2L = _layer_call(u1L, i1L, gA, dA, vA, gB, dB, vB,
                           gC, dC, vC, gD, dD, vD)

    user_out = (user_emb + _from_half_layout(u1L, NU) + _from_half_layout(u2L, NU)) / 3.0
    item_out = (item_emb + _from_half_layout(i1L, NI) + _from_half_layout(i2L, NI)) / 3.0
    return (user_out, item_out)


# speedup vs baseline: 6.1144x; 2.5542x over previous
import functools

import jax
import jax.numpy as jnp
from jax import lax
from jax.experimental import pallas as pl
from jax.experimental.pallas import tpu as pltpu
from jax.experimental.pallas import tpu_sc as plsc

NU = 10000
NI = 10000
D = 256
E = 160000
HALF = 128
K = 112
NCHUNK = -(-E // K)
EPAD = NCHUNK * K - E
NSUB = 16
PN = 10240
ROWS_PER_TEC = PN // NSUB


def _to_half_layout(x):
    n = x.shape[0]
    h = x.reshape(n, 2, HALF).transpose(1, 0, 2)
    h = jnp.pad(h, ((0, 0), (0, PN - n), (0, 0)))
    return h.reshape(2 * PN, HALF)


def _from_half_layout(x, n):
    return x.reshape(2, PN, HALF)[:, :n].transpose(1, 0, 2).reshape(n, D)


def _prep_edges(idx, val):
    idx = idx.astype(jnp.int32)
    src = jnp.pad(idx[1], (0, EPAD))
    dst = jnp.pad(idx[0], (0, EPAD))
    vals = jnp.pad(val, (0, EPAD))
    gidx = jnp.stack([src, src + PN]).reshape(2, NCHUNK, 1, K)
    didx = dst.reshape(NCHUNK, 1, K)
    vals = vals.reshape(NCHUNK, 1, K)
    return gidx, didx, vals


def _layer_body(uL, iL,
                g1, d1, v1,
                g2, d2, v2,
                g3, d3, v3,
                g4, d4, v4,
                u_out, i_out,
                acc, gidx_v, didx_v, sdix_v, val_v, rows_v,
                idx_sem, gat_sem, scat_sem):
    c = lax.axis_index("c")
    s = lax.axis_index("s")

    def zero_own_rows():
        def zrow(i, carry):
            for q in range(8):
                rows_v[0, i, pl.ds(q * 16, 16)] = jnp.zeros((16,), jnp.float32)
            return carry

        lax.fori_loop(0, 64, zrow, 0)
        zsrc = rows_v.at[0, pl.ds(0, 64)]

        @pl.when(s < NSUB - 1)
        def _():
            for k in range(10):
                pltpu.sync_copy(zsrc, acc.at[pl.ds(s * ROWS_PER_TEC + k * 64, 64)])

        @pl.when(s == NSUB - 1)
        def _():
            for k in range(6):
                pltpu.sync_copy(zsrc, acc.at[pl.ds(9600 + k * 64, 64)])
            pltpu.sync_copy(rows_v.at[0, pl.ds(0, 16)], acc.at[pl.ds(9984, 16)])

    NW = (NCHUNK + NSUB - 1) // NSUB

    def spmm(g, d, v, table):
        def tof(i):
            return s + NSUB * i

        def issue_idx(i, b):
            pltpu.async_copy(g.at[c, tof(i), 0], gidx_v.at[b], idx_sem.at[b])
            pltpu.async_copy(d.at[tof(i), 0], didx_v.at[b], idx_sem.at[b])
            pltpu.async_copy(v.at[tof(i), 0], val_v.at[b], idx_sem.at[b])

        def wait_idx(i, b):
            pltpu.make_async_copy(g.at[c, tof(i), 0], gidx_v.at[b], idx_sem.at[b]).wait()
            pltpu.make_async_copy(d.at[tof(i), 0], didx_v.at[b], idx_sem.at[b]).wait()
            pltpu.make_async_copy(v.at[tof(i), 0], val_v.at[b], idx_sem.at[b]).wait()

        def issue_gather(i, b):
            pltpu.async_copy(table.at[gidx_v.at[b]], rows_v.at[b], gat_sem.at[b])

        def wait_gather(i, b):
            pltpu.make_async_copy(table.at[gidx_v.at[b]], rows_v.at[b], gat_sem.at[b]).wait()

        def issue_scat(b):
            pltpu.async_copy(rows_v.at[b], acc.at[sdix_v.at[b]], scat_sem.at[b], add=True)

        def wait_scat(b):
            pltpu.make_async_copy(rows_v.at[b], acc.at[sdix_v.at[b]], scat_sem.at[b]).wait()

        issue_idx(0, 0)
        issue_idx(1, 1)
        wait_idx(0, 0)
        issue_gather(0, 0)

        NG = (NW + 2) // 3

        def group(grp, carry):
            for b0 in range(3):
                i = grp * 3 + b0
                b = b0
                bn = (b0 + 1) % 3
                bn2 = (b0 + 2) % 3

                @pl.when(tof(i + 1) < NCHUNK)
                def _(i=i, bn=bn):
                    wait_idx(i + 1, bn)

                    @pl.when(i >= 2)
                    def _():
                        wait_scat(bn)

                    issue_gather(i + 1, bn)

                @pl.when(tof(i + 2) < NCHUNK)
                def _(i=i, bn2=bn2):
                    issue_idx(i + 2, bn2)

                @pl.when(tof(i) < NCHUNK)
                def _(i=i, b=b):
                    wait_gather(i, b)
                    for q in range(K // 16):
                        sl = pl.ds(q * 16, 16)
                        sdix_v[b, sl] = didx_v[b, sl]

                    def scale(jb, carry2):
                        vv = val_v[b, pl.ds(jb * 16, 16)]
                        for l in range(16):
                            vj = vv[l]
                            j = jb * 16 + l
                            for q in range(HALF // 16):
                                sl = pl.ds(q * 16, 16)
                                rows_v[b, j, sl] = rows_v[b, j, sl] * vj
                        return carry2

                    lax.fori_loop(0, K // 16, scale, 0)
                    issue_scat(b)

            return carry

        lax.fori_loop(0, NG, group, 0)

        for j in range(NW - 4, NW):
            @pl.when(jnp.logical_and(tof(j) < NCHUNK, tof(j + 3) >= NCHUNK))
            def _(j=j):
                wait_scat(j % 3)

    def flush(out_ref):
        @pl.when(s < NSUB - 1)
        def _():
            pltpu.sync_copy(acc.at[pl.ds(s * ROWS_PER_TEC, ROWS_PER_TEC)],
                            out_ref.at[pl.ds(c * PN + s * ROWS_PER_TEC, ROWS_PER_TEC)])

        @pl.when(s == NSUB - 1)
        def _():
            pltpu.sync_copy(acc.at[pl.ds(9600, 400)],
                            out_ref.at[pl.ds(c * PN + 9600, 400)])

    zero_own_rows()
    plsc.subcore_barrier()
    spmm(g1, d1, v1, uL)
    spmm(g2, d2, v2, iL)
    plsc.subcore_barrier()
    flush(u_out)
    zero_own_rows()
    plsc.subcore_barrier()
    spmm(g3, d3, v3, iL)
    spmm(g4, d4, v4, uL)
    plsc.subcore_barrier()
    flush(i_out)


_layer_call = functools.partial(
    pl.kernel,
    mesh=plsc.VectorSubcoreMesh(core_axis_name="c", subcore_axis_name="s"),
    out_type=[
        jax.ShapeDtypeStruct((2 * PN, HALF), jnp.float32),
        jax.ShapeDtypeStruct((2 * PN, HALF), jnp.float32),
    ],
    scratch_types=[
        pltpu.VMEM_SHARED((NU, HALF), jnp.float32),
        pltpu.VMEM((3, K), jnp.int32),
        pltpu.VMEM((3, K), jnp.int32),
        pltpu.VMEM((3, K), jnp.int32),
        pltpu.VMEM((3, K), jnp.float32),
        pltpu.VMEM((3, K, HALF), jnp.float32),
        pltpu.SemaphoreType.DMA((3,)),
        pltpu.SemaphoreType.DMA((3,)),
        pltpu.SemaphoreType.DMA((3,)),
    ],
)(_layer_body)


@jax.jit
def kernel(user_emb, item_emb, u2u_idx, u2u_val, u2i_idx, u2i_val,
           i2u_idx, i2u_val, i2i_idx, i2i_val):
    uL = _to_half_layout(user_emb)
    iL = _to_half_layout(item_emb)

    gA, dA, vA = _prep_edges(u2u_idx, u2u_val)
    gB, dB, vB = _prep_edges(u2i_idx, u2i_val)
    gC, dC, vC = _prep_edges(i2i_idx, i2i_val)
    gD, dD, vD = _prep_edges(i2u_idx, i2u_val)

    u1L, i1L = _layer_call(uL, iL, gA, dA, vA, gB, dB, vB,
                           gC, dC, vC, gD, dD, vD)
    u2L, i2L = _layer_call(u1L, i1L, gA, dA, vA, gB, dB, vB,
                           gC, dC, vC, gD, dD, vD)

    user_out = (user_emb + _from_half_layout(u1L, NU) + _from_half_layout(u2L, NU)) / 3.0
    item_out = (item_emb + _from_half_layout(i1L, NI) + _from_half_layout(i2L, NI)) / 3.0
    return (user_out, item_out)
